# baseline (speedup 1.0000x reference)
import functools

import jax
import jax.numpy as jnp
from jax import lax
from jax.experimental import pallas as pl
from jax.experimental.pallas import tpu as pltpu
from jax.experimental.pallas import tpu_sc as plsc

_NC = 2
_NS = 16
_TILE = 256


def _router_body(x_ref, gw_ref, gb_ref, e_ref, w_ref, psum_ref, loss_ref, *,
                 n_tokens, n_experts):
    t = pl.program_id(0)
    xs = x_ref[...]
    gw = gw_ref[...]
    logits = lax.dot_general(xs, gw, (((1,), (1,)), ((), ())),
                             preferred_element_type=jnp.float32) + gb_ref[...]
    m = jnp.max(logits, axis=1, keepdims=True)
    p = jnp.exp(logits - m)
    probs = p / jnp.sum(p, axis=1, keepdims=True)
    iota = lax.broadcasted_iota(jnp.int32, probs.shape, 1)
    m1 = jnp.max(probs, axis=1, keepdims=True)
    i0 = jnp.min(jnp.where(probs == m1, iota, n_experts), axis=1, keepdims=True)
    probs2 = jnp.where(iota == i0, -1.0, probs)
    m2 = jnp.max(probs2, axis=1, keepdims=True)
    i1 = jnp.min(jnp.where(probs2 == m2, iota, n_experts), axis=1, keepdims=True)
    e_ref[...] = jnp.concatenate([i0, i1], axis=1)
    w_ref[...] = jnp.concatenate([m1, m2], axis=1)

    @pl.when(t == 0)
    def _():
        psum_ref[...] = jnp.zeros_like(psum_ref)

    psum_ref[...] += jnp.sum(probs, axis=0, keepdims=True)

    @pl.when(t == pl.num_programs(0) - 1)
    def _():
        mean = psum_ref[...] * (1.0 / n_tokens)
        loss_ref[...] = jnp.sum(mean * mean, axis=1, keepdims=True) * n_experts


def _route_pos_body(e_ref, pos_ref, te_ref, *, n_assign, n_experts, n_tiles):
    blk = 128
    n_rows = n_assign // blk
    ev = e_ref[...]
    rb = lax.broadcasted_iota(jnp.int32, (blk, blk), 0)
    cb = lax.broadcasted_iota(jnp.int32, (blk, blk), 1)
    ustrict = (rb < cb).astype(jnp.float32)
    rr = lax.broadcasted_iota(jnp.int32, (n_rows, n_rows), 0)
    cr = lax.broadcasted_iota(jnp.int32, (n_rows, n_rows), 1)
    lstrict = (cr < rr).astype(jnp.float32)
    iota_e = lax.broadcasted_iota(jnp.int32, (1, n_experts), 1)

    rank = jnp.zeros((n_rows, blk), jnp.float32)
    totals = jnp.zeros((1, n_experts), jnp.float32)
    ms = []
    for e in range(n_experts):
        m = (ev == e).astype(jnp.float32)
        ms.append(m)
        lane_pre = lax.dot_general(m, ustrict, (((1,), (0,)), ((), ())),
                                   preferred_element_type=jnp.float32)
        s = jnp.sum(m, axis=1, keepdims=True)
        row_pre = lax.dot_general(lstrict, s, (((1,), (0,)), ((), ())),
                                  preferred_element_type=jnp.float32)
        rank = rank + m * (lane_pre + row_pre)
        totals = totals + jnp.sum(s, axis=0, keepdims=True) * (
            iota_e == e).astype(jnp.float32)

    padded = jnp.ceil(totals * (1.0 / _TILE)) * _TILE
    re = lax.broadcasted_iota(jnp.int32, (n_experts, n_experts), 0)
    ce = lax.broadcasted_iota(jnp.int32, (n_experts, n_experts), 1)
    lower_inc = (re <= ce).astype(jnp.float32)
    incl = lax.dot_general(padded, lower_inc, (((1,), (0,)), ((), ())),
                           preferred_element_type=jnp.float32)
    base = incl - padded

    basesel = jnp.zeros((n_rows, blk), jnp.float32)
    for e in range(n_experts):
        basesel = basesel + ms[e] * base[0:1, e:e + 1]
    pos_ref[...] = (rank + basesel).astype(jnp.int32)

    t_ids = lax.broadcasted_iota(jnp.int32, (1, 128), 1)
    end_tiles = (incl * (1.0 / _TILE)).astype(jnp.int32)
    te = jnp.zeros((1, 128), jnp.int32)
    for e in range(n_experts):
        end_e = end_tiles[0:1, e:e + 1]
        te = te + (t_ids >= end_e).astype(jnp.int32)
    te_ref[...] = jnp.minimum(te, n_experts - 1)


def _sc_scatter_body(x_hbm, pos_hbm, xs_hbm, posv, xbuf, sem):
    wid = lax.axis_index("s") * _NC + lax.axis_index("c")
    tb = lax.rem(wid * 256, 4096)
    pltpu.sync_copy(pos_hbm.at[pl.ds(wid * 4, 4)], posv)
    for c in range(4):
        pltpu.sync_copy(x_hbm.at[pl.ds(tb + c * 64, 64)], xbuf)
        pltpu.async_copy(xbuf, xs_hbm.at[posv.at[c]], sem).wait()


def _sc_gather_body(ys_hbm, pos_hbm, rows_hbm, posv, xbuf, sem):
    wid = lax.axis_index("s") * _NC + lax.axis_index("c")
    pltpu.sync_copy(pos_hbm.at[pl.ds(wid * 4, 4)], posv)
    for c in range(4):
        pltpu.async_copy(ys_hbm.at[posv.at[c]], xbuf, sem).wait()
        pltpu.sync_copy(xbuf, rows_hbm.at[pl.ds(wid * 256 + c * 64, 64)])


def _ffn_sparse_body(te_ref, xs_ref, w1_ref, b1_ref, w2_ref, b2_ref, out_ref):
    del te_ref
    h = lax.dot_general(xs_ref[...], w1_ref[0], (((1,), (1,)), ((), ())),
                        preferred_element_type=jnp.float32)
    h = jnp.maximum(h + b1_ref[0], 0.0).astype(jnp.bfloat16)
    proc = lax.dot_general(h, w2_ref[0], (((1,), (1,)), ((), ())),
                           preferred_element_type=jnp.float32)
    out_ref[...] = (proc + b2_ref[0]).astype(jnp.bfloat16)


def _combine_body(y0_ref, y1_ref, w0_ref, w1_ref, out_ref):
    out_ref[...] = (y0_ref[...].astype(jnp.float32) * w0_ref[...] +
                    y1_ref[...].astype(jnp.float32) * w1_ref[...])


def kernel(x, gate_W, gate_b, W1, b1, W2, b2):
    seq_len, batch, d = x.shape
    n_experts, dff, _ = W1.shape
    tokens = seq_len * batch
    n_assign = 2 * tokens
    cap = n_assign + n_experts * _TILE
    n_tiles = cap // _TILE
    x_flat = x.reshape(tokens, d)

    rt = 512
    ew, wv, _, loss = pl.pallas_call(
        functools.partial(_router_body, n_tokens=tokens, n_experts=n_experts),
        grid=(tokens // rt,),
        in_specs=[
            pl.BlockSpec((rt, d), lambda t: (t, 0)),
            pl.BlockSpec((n_experts, d), lambda t: (0, 0)),
            pl.BlockSpec((1, n_experts), lambda t: (0, 0)),
        ],
        out_specs=[
            pl.BlockSpec((rt, 2), lambda t: (t, 0)),
            pl.BlockSpec((rt, 2), lambda t: (t, 0)),
            pl.BlockSpec((1, n_experts), lambda t: (0, 0)),
            pl.BlockSpec((1, 1), lambda t: (0, 0)),
        ],
        out_shape=[
            jax.ShapeDtypeStruct((tokens, 2), jnp.int32),
            jax.ShapeDtypeStruct((tokens, 2), jnp.float32),
            jax.ShapeDtypeStruct((1, n_experts), jnp.float32),
            jax.ShapeDtypeStruct((1, 1), jnp.float32),
        ],
    )(x_flat, gate_W, gate_b.reshape(1, n_experts))

    eflat = ew.T.reshape(n_assign // 128, 128)

    pos, te = pl.pallas_call(
        functools.partial(_route_pos_body, n_assign=n_assign,
                          n_experts=n_experts, n_tiles=n_tiles),
        out_shape=[
            jax.ShapeDtypeStruct((n_assign // 128, 128), jnp.int32),
            jax.ShapeDtypeStruct((1, 128), jnp.int32),
        ],
    )(eflat)

    pos2d = pos.reshape(128, 64)

    dh = d // 2
    x_bfv = lax.bitcast_convert_type(
        x_flat.astype(jnp.bfloat16).reshape(tokens, dh, 2), jnp.float32)
    mesh = plsc.VectorSubcoreMesh(core_axis_name="c", subcore_axis_name="s")
    xs_v = pl.kernel(
        _sc_scatter_body,
        out_type=jax.ShapeDtypeStruct((cap, dh), jnp.float32),
        mesh=mesh,
        scratch_types=[
            pltpu.VMEM((4, 64), jnp.int32),
            pltpu.VMEM((64, dh), jnp.float32),
            pltpu.SemaphoreType.DMA,
        ],
    )(x_bfv, pos2d)
    xs_sorted = lax.bitcast_convert_type(xs_v, jnp.bfloat16).reshape(cap, d)

    grid_spec = pltpu.PrefetchScalarGridSpec(
        num_scalar_prefetch=1,
        grid=(n_tiles,),
        in_specs=[
            pl.BlockSpec((_TILE, d), lambda t, te_r: (t, 0)),
            pl.BlockSpec((1, dff, d), lambda t, te_r: (te_r[0, t], 0, 0)),
            pl.BlockSpec((1, 1, dff), lambda t, te_r: (te_r[0, t], 0, 0)),
            pl.BlockSpec((1, d, dff), lambda t, te_r: (te_r[0, t], 0, 0)),
            pl.BlockSpec((1, 1, d), lambda t, te_r: (te_r[0, t], 0, 0)),
        ],
        out_specs=pl.BlockSpec((_TILE, d), lambda t, te_r: (t, 0)),
    )
    ys_sorted = pl.pallas_call(
        _ffn_sparse_body,
        grid_spec=grid_spec,
        out_shape=jax.ShapeDtypeStruct((cap, d), jnp.bfloat16),
    )(te, xs_sorted, W1.astype(jnp.bfloat16), b1.reshape(n_experts, 1, dff),
      W2.astype(jnp.bfloat16), b2.reshape(n_experts, 1, d))

    ys_v = lax.bitcast_convert_type(ys_sorted.reshape(cap, dh, 2), jnp.float32)
    rows_v = pl.kernel(
        _sc_gather_body,
        out_type=jax.ShapeDtypeStruct((n_assign, dh), jnp.float32),
        mesh=mesh,
        scratch_types=[
            pltpu.VMEM((4, 64), jnp.int32),
            pltpu.VMEM((64, dh), jnp.float32),
            pltpu.SemaphoreType.DMA,
        ],
    )(ys_v, pos2d)
    rows = lax.bitcast_convert_type(rows_v, jnp.bfloat16).reshape(n_assign, d)

    ct = 512
    out_flat = pl.pallas_call(
        _combine_body,
        grid=(tokens // ct,),
        in_specs=[
            pl.BlockSpec((ct, d), lambda t: (t, 0)),
            pl.BlockSpec((ct, d), lambda t: (t + tokens // ct, 0)),
            pl.BlockSpec((ct, 1), lambda t: (t, 0)),
            pl.BlockSpec((ct, 1), lambda t: (t, 0)),
        ],
        out_specs=pl.BlockSpec((ct, d), lambda t: (t, 0)),
        out_shape=jax.ShapeDtypeStruct((tokens, d), jnp.float32),
    )(rows, rows, wv[:, 0:1], wv[:, 1:2])

    return (out_flat.reshape(seq_len, batch, d), loss.reshape(()))


# speedup vs baseline: 5.1729x; 5.1729x over previous
import functools

import jax
import jax.numpy as jnp
from jax import lax
from jax.experimental import pallas as pl
from jax.experimental.pallas import tpu as pltpu
from jax.experimental.pallas import tpu_sc as plsc

_NC = 2
_NS = 16
_TILE = 256


def _router_body(x_ref, gw_ref, gb_ref, e_ref, w_ref, psum_ref, loss_ref, *,
                 n_tokens, n_experts):
    t = pl.program_id(0)
    xs = x_ref[...]
    gw = gw_ref[...]
    logits = lax.dot_general(xs, gw, (((1,), (1,)), ((), ())),
                             preferred_element_type=jnp.float32) + gb_ref[...]
    m = jnp.max(logits, axis=1, keepdims=True)
    p = jnp.exp(logits - m)
    probs = p / jnp.sum(p, axis=1, keepdims=True)
    iota = lax.broadcasted_iota(jnp.int32, probs.shape, 1)
    m1 = jnp.max(probs, axis=1, keepdims=True)
    i0 = jnp.min(jnp.where(probs == m1, iota, n_experts), axis=1, keepdims=True)
    probs2 = jnp.where(iota == i0, -1.0, probs)
    m2 = jnp.max(probs2, axis=1, keepdims=True)
    i1 = jnp.min(jnp.where(probs2 == m2, iota, n_experts), axis=1, keepdims=True)
    e_ref[...] = jnp.concatenate([i0, i1], axis=1)
    w_ref[...] = jnp.concatenate([m1, m2], axis=1)

    @pl.when(t == 0)
    def _():
        psum_ref[...] = jnp.zeros_like(psum_ref)

    psum_ref[...] += jnp.sum(probs, axis=0, keepdims=True)

    @pl.when(t == pl.num_programs(0) - 1)
    def _():
        mean = psum_ref[...] * (1.0 / n_tokens)
        loss_ref[...] = jnp.sum(mean * mean, axis=1, keepdims=True) * n_experts


def _route_pos_body(e_ref, pos_ref, te_ref, *, n_assign, n_experts, n_tiles):
    blk = 128
    n_rows = n_assign // blk
    ev = e_ref[...]
    rb = lax.broadcasted_iota(jnp.int32, (blk, blk), 0)
    cb = lax.broadcasted_iota(jnp.int32, (blk, blk), 1)
    ustrict = (rb < cb).astype(jnp.float32)
    rr = lax.broadcasted_iota(jnp.int32, (n_rows, n_rows), 0)
    cr = lax.broadcasted_iota(jnp.int32, (n_rows, n_rows), 1)
    lstrict = (cr < rr).astype(jnp.float32)
    iota_e = lax.broadcasted_iota(jnp.int32, (1, n_experts), 1)

    rank = jnp.zeros((n_rows, blk), jnp.float32)
    totals = jnp.zeros((1, n_experts), jnp.float32)
    ms = []
    for e in range(n_experts):
        m = (ev == e).astype(jnp.float32)
        ms.append(m)
        lane_pre = lax.dot_general(m, ustrict, (((1,), (0,)), ((), ())),
                                   preferred_element_type=jnp.float32)
        s = jnp.sum(m, axis=1, keepdims=True)
        row_pre = lax.dot_general(lstrict, s, (((1,), (0,)), ((), ())),
                                  preferred_element_type=jnp.float32)
        rank = rank + m * (lane_pre + row_pre)
        totals = totals + jnp.sum(s, axis=0, keepdims=True) * (
            iota_e == e).astype(jnp.float32)

    padded = jnp.ceil(totals * (1.0 / _TILE)) * _TILE
    re = lax.broadcasted_iota(jnp.int32, (n_experts, n_experts), 0)
    ce = lax.broadcasted_iota(jnp.int32, (n_experts, n_experts), 1)
    lower_inc = (re <= ce).astype(jnp.float32)
    incl = lax.dot_general(padded, lower_inc, (((1,), (0,)), ((), ())),
                           preferred_element_type=jnp.float32)
    base = incl - padded

    basesel = jnp.zeros((n_rows, blk), jnp.float32)
    for e in range(n_experts):
        basesel = basesel + ms[e] * base[0:1, e:e + 1]
    pos_ref[...] = (rank + basesel).astype(jnp.int32)

    t_ids = lax.broadcasted_iota(jnp.int32, (1, 128), 1)
    end_tiles = (incl * (1.0 / _TILE)).astype(jnp.int32)
    te = jnp.zeros((1, 128), jnp.int32)
    for e in range(n_experts):
        end_e = end_tiles[0:1, e:e + 1]
        te = te + (t_ids >= end_e).astype(jnp.int32)
    te_ref[...] = jnp.minimum(te, n_experts - 1)


def _sc_scatter_body(x_hbm, pos_hbm, xs_hbm, posv, xbuf, sem):
    wid = lax.axis_index("s") * _NC + lax.axis_index("c")
    tb = lax.rem(wid * 256, 4096)
    pltpu.sync_copy(pos_hbm.at[pl.ds(wid * 4, 4)], posv)
    for c in range(4):
        pltpu.sync_copy(x_hbm.at[pl.ds(tb + c * 64, 64)], xbuf)
        pltpu.async_copy(xbuf, xs_hbm.at[posv.at[c]], sem).wait()


def _sc_gather_body(ys_hbm, pos_hbm, rows_hbm, posv, xbuf, sem):
    wid = lax.axis_index("s") * _NC + lax.axis_index("c")
    pltpu.sync_copy(pos_hbm.at[pl.ds(wid * 4, 4)], posv)
    for c in range(4):
        pltpu.async_copy(ys_hbm.at[posv.at[c]], xbuf, sem).wait()
        pltpu.sync_copy(xbuf, rows_hbm.at[pl.ds(wid * 256 + c * 64, 64)])


def _ffn_sparse_body(te_ref, xs_ref, w1_ref, b1_ref, w2_ref, b2_ref, out_ref):
    del te_ref
    h = lax.dot_general(xs_ref[...].astype(jnp.bfloat16), w1_ref[0],
                        (((1,), (1,)), ((), ())),
                        preferred_element_type=jnp.float32)
    h = jnp.maximum(h + b1_ref[0], 0.0).astype(jnp.bfloat16)
    proc = lax.dot_general(h, w2_ref[0], (((1,), (1,)), ((), ())),
                           preferred_element_type=jnp.float32)
    out_ref[...] = proc + b2_ref[0]


def _combine_body(y0_ref, y1_ref, w0_ref, w1_ref, out_ref):
    out_ref[...] = y0_ref[...] * w0_ref[...] + y1_ref[...] * w1_ref[...]


def kernel(x, gate_W, gate_b, W1, b1, W2, b2):
    seq_len, batch, d = x.shape
    n_experts, dff, _ = W1.shape
    tokens = seq_len * batch
    n_assign = 2 * tokens
    cap = n_assign + n_experts * _TILE
    n_tiles = cap // _TILE
    x_flat = x.reshape(tokens, d)

    rt = 512
    ew, wv, _, loss = pl.pallas_call(
        functools.partial(_router_body, n_tokens=tokens, n_experts=n_experts),
        grid=(tokens // rt,),
        in_specs=[
            pl.BlockSpec((rt, d), lambda t: (t, 0)),
            pl.BlockSpec((n_experts, d), lambda t: (0, 0)),
            pl.BlockSpec((1, n_experts), lambda t: (0, 0)),
        ],
        out_specs=[
            pl.BlockSpec((rt, 2), lambda t: (t, 0)),
            pl.BlockSpec((rt, 2), lambda t: (t, 0)),
            pl.BlockSpec((1, n_experts), lambda t: (0, 0)),
            pl.BlockSpec((1, 1), lambda t: (0, 0)),
        ],
        out_shape=[
            jax.ShapeDtypeStruct((tokens, 2), jnp.int32),
            jax.ShapeDtypeStruct((tokens, 2), jnp.float32),
            jax.ShapeDtypeStruct((1, n_experts), jnp.float32),
            jax.ShapeDtypeStruct((1, 1), jnp.float32),
        ],
    )(x_flat, gate_W, gate_b.reshape(1, n_experts))

    eflat = ew.T.reshape(n_assign // 128, 128)

    pos, te = pl.pallas_call(
        functools.partial(_route_pos_body, n_assign=n_assign,
                          n_experts=n_experts, n_tiles=n_tiles),
        out_shape=[
            jax.ShapeDtypeStruct((n_assign // 128, 128), jnp.int32),
            jax.ShapeDtypeStruct((1, 128), jnp.int32),
        ],
    )(eflat)

    pos2d = pos.reshape(128, 64)

    mesh = plsc.VectorSubcoreMesh(core_axis_name="c", subcore_axis_name="s")
    xs_sorted = pl.kernel(
        _sc_scatter_body,
        out_type=jax.ShapeDtypeStruct((cap, d), jnp.float32),
        mesh=mesh,
        scratch_types=[
            pltpu.VMEM((4, 64), jnp.int32),
            pltpu.VMEM((64, d), jnp.float32),
            pltpu.SemaphoreType.DMA,
        ],
    )(x_flat, pos2d)

    grid_spec = pltpu.PrefetchScalarGridSpec(
        num_scalar_prefetch=1,
        grid=(n_tiles,),
        in_specs=[
            pl.BlockSpec((_TILE, d), lambda t, te_r: (t, 0)),
            pl.BlockSpec((1, dff, d), lambda t, te_r: (te_r[0, t], 0, 0)),
            pl.BlockSpec((1, 1, dff), lambda t, te_r: (te_r[0, t], 0, 0)),
            pl.BlockSpec((1, d, dff), lambda t, te_r: (te_r[0, t], 0, 0)),
            pl.BlockSpec((1, 1, d), lambda t, te_r: (te_r[0, t], 0, 0)),
        ],
        out_specs=pl.BlockSpec((_TILE, d), lambda t, te_r: (t, 0)),
    )
    ys_sorted = pl.pallas_call(
        _ffn_sparse_body,
        grid_spec=grid_spec,
        out_shape=jax.ShapeDtypeStruct((cap, d), jnp.float32),
    )(te, xs_sorted, W1.astype(jnp.bfloat16), b1.reshape(n_experts, 1, dff),
      W2.astype(jnp.bfloat16), b2.reshape(n_experts, 1, d))

    rows = pl.kernel(
        _sc_gather_body,
        out_type=jax.ShapeDtypeStruct((n_assign, d), jnp.float32),
        mesh=mesh,
        scratch_types=[
            pltpu.VMEM((4, 64), jnp.int32),
            pltpu.VMEM((64, d), jnp.float32),
            pltpu.SemaphoreType.DMA,
        ],
    )(ys_sorted, pos2d)

    ct = 512
    out_flat = pl.pallas_call(
        _combine_body,
        grid=(tokens // ct,),
        in_specs=[
            pl.BlockSpec((ct, d), lambda t: (t, 0)),
            pl.BlockSpec((ct, d), lambda t: (t + tokens // ct, 0)),
            pl.BlockSpec((ct, 1), lambda t: (t, 0)),
            pl.BlockSpec((ct, 1), lambda t: (t, 0)),
        ],
        out_specs=pl.BlockSpec((ct, d), lambda t: (t, 0)),
        out_shape=jax.ShapeDtypeStruct((tokens, d), jnp.float32),
    )(rows, rows, wv[:, 0:1], wv[:, 1:2])

    return (out_flat.reshape(seq_len, batch, d), loss.reshape(()))


# speedup vs baseline: 5.3027x; 1.0251x over previous
import functools

import jax
import jax.numpy as jnp
from jax import lax
from jax.experimental import pallas as pl
from jax.experimental.pallas import tpu as pltpu
from jax.experimental.pallas import tpu_sc as plsc

_NC = 2
_NS = 16
_TILE = 256


def _router_body(x_ref, gw_ref, gb_ref, e_ref, w_ref, psum_ref, loss_ref, *,
                 n_tokens, n_experts):
    t = pl.program_id(0)
    xs = x_ref[...]
    gw = gw_ref[...]
    logits = lax.dot_general(xs, gw, (((1,), (1,)), ((), ())),
                             preferred_element_type=jnp.float32) + gb_ref[...]
    m = jnp.max(logits, axis=1, keepdims=True)
    p = jnp.exp(logits - m)
    probs = p / jnp.sum(p, axis=1, keepdims=True)
    iota = lax.broadcasted_iota(jnp.int32, probs.shape, 1)
    m1 = jnp.max(probs, axis=1, keepdims=True)
    i0 = jnp.min(jnp.where(probs == m1, iota, n_experts), axis=1, keepdims=True)
    probs2 = jnp.where(iota == i0, -1.0, probs)
    m2 = jnp.max(probs2, axis=1, keepdims=True)
    i1 = jnp.min(jnp.where(probs2 == m2, iota, n_experts), axis=1, keepdims=True)
    e_ref[...] = jnp.concatenate([i0, i1], axis=1)
    w_ref[...] = jnp.concatenate([m1, m2], axis=1)

    @pl.when(t == 0)
    def _():
        psum_ref[...] = jnp.zeros_like(psum_ref)

    psum_ref[...] += jnp.sum(probs, axis=0, keepdims=True)

    @pl.when(t == pl.num_programs(0) - 1)
    def _():
        mean = psum_ref[...] * (1.0 / n_tokens)
        loss_ref[...] = jnp.sum(mean * mean, axis=1, keepdims=True) * n_experts


def _route_pos_body(e_ref, pos_ref, te_ref, *, n_assign, n_experts, n_tiles):
    blk = 128
    n_rows = n_assign // blk
    ev = e_ref[...]
    rb = lax.broadcasted_iota(jnp.int32, (blk, blk), 0)
    cb = lax.broadcasted_iota(jnp.int32, (blk, blk), 1)
    ustrict = (rb < cb).astype(jnp.float32)
    rr = lax.broadcasted_iota(jnp.int32, (n_rows, n_rows), 0)
    cr = lax.broadcasted_iota(jnp.int32, (n_rows, n_rows), 1)
    lstrict = (cr < rr).astype(jnp.float32)
    iota_e = lax.broadcasted_iota(jnp.int32, (1, n_experts), 1)

    rank = jnp.zeros((n_rows, blk), jnp.float32)
    totals = jnp.zeros((1, n_experts), jnp.float32)
    ms = []
    for e in range(n_experts):
        m = (ev == e).astype(jnp.float32)
        ms.append(m)
        lane_pre = lax.dot_general(m, ustrict, (((1,), (0,)), ((), ())),
                                   preferred_element_type=jnp.float32)
        s = jnp.sum(m, axis=1, keepdims=True)
        row_pre = lax.dot_general(lstrict, s, (((1,), (0,)), ((), ())),
                                  preferred_element_type=jnp.float32)
        rank = rank + m * (lane_pre + row_pre)
        totals = totals + jnp.sum(s, axis=0, keepdims=True) * (
            iota_e == e).astype(jnp.float32)

    padded = jnp.ceil(totals * (1.0 / _TILE)) * _TILE
    re = lax.broadcasted_iota(jnp.int32, (n_experts, n_experts), 0)
    ce = lax.broadcasted_iota(jnp.int32, (n_experts, n_experts), 1)
    lower_inc = (re <= ce).astype(jnp.float32)
    incl = lax.dot_general(padded, lower_inc, (((1,), (0,)), ((), ())),
                           preferred_element_type=jnp.float32)
    base = incl - padded

    basesel = jnp.zeros((n_rows, blk), jnp.float32)
    for e in range(n_experts):
        basesel = basesel + ms[e] * base[0:1, e:e + 1]
    pos_ref[...] = (rank + basesel).astype(jnp.int32)

    t_ids = lax.broadcasted_iota(jnp.int32, (1, 128), 1)
    end_tiles = (incl * (1.0 / _TILE)).astype(jnp.int32)
    te = jnp.zeros((1, 128), jnp.int32)
    for e in range(n_experts):
        end_e = end_tiles[0:1, e:e + 1]
        te = te + (t_ids >= end_e).astype(jnp.int32)
    te_ref[...] = jnp.minimum(te, n_experts - 1)


def _sc_scatter_body(x_hbm, pos_hbm, xs_hbm, posv, xbuf, sem):
    wid = lax.axis_index("s") * _NC + lax.axis_index("c")
    tb = lax.rem(wid * 256, 4096)
    pltpu.sync_copy(pos_hbm.at[pl.ds(wid * 4, 4)], posv)
    for c in range(4):
        pltpu.sync_copy(x_hbm.at[pl.ds(tb + c * 64, 64)], xbuf)
        pltpu.async_copy(xbuf, xs_hbm.at[posv.at[c]], sem).wait()


def _sc_gather_body(ya_hbm, yb_hbm, pos_hbm, ra_hbm, rb_hbm, posv, xbuf, sem):
    wid = lax.axis_index("s") * _NC + lax.axis_index("c")
    pltpu.sync_copy(pos_hbm.at[pl.ds(wid * 4, 4)], posv)
    for c in range(4):
        pltpu.async_copy(ya_hbm.at[posv.at[c]], xbuf, sem).wait()
        pltpu.sync_copy(xbuf, ra_hbm.at[pl.ds(wid * 256 + c * 64, 64)])
        pltpu.async_copy(yb_hbm.at[posv.at[c]], xbuf, sem).wait()
        pltpu.sync_copy(xbuf, rb_hbm.at[pl.ds(wid * 256 + c * 64, 64)])


def _ffn_half_body(te_ref, xs_ref, w1_ref, b1_ref, w2_ref, b2_ref, out_ref, *,
                   use_b2):
    del te_ref
    h = lax.dot_general(xs_ref[...], w1_ref[0], (((1,), (1,)), ((), ())),
                        preferred_element_type=jnp.float32)
    h = jnp.maximum(h + b1_ref[0], 0.0)
    proc = lax.dot_general(h, w2_ref[0], (((1,), (1,)), ((), ())),
                           preferred_element_type=jnp.float32)
    out_ref[...] = proc + b2_ref[0] if use_b2 else proc


def _combine_body(a0_ref, a1_ref, b0_ref, b1_ref, w0_ref, w1_ref, out_ref):
    out_ref[...] = ((a0_ref[...] + b0_ref[...]) * w0_ref[...] +
                    (a1_ref[...] + b1_ref[...]) * w1_ref[...])


def kernel(x, gate_W, gate_b, W1, b1, W2, b2):
    seq_len, batch, d = x.shape
    n_experts, dff, _ = W1.shape
    tokens = seq_len * batch
    n_assign = 2 * tokens
    cap = n_assign + n_experts * _TILE
    n_tiles = cap // _TILE
    x_flat = x.reshape(tokens, d)

    rt = 512
    ew, wv, _, loss = pl.pallas_call(
        functools.partial(_router_body, n_tokens=tokens, n_experts=n_experts),
        grid=(tokens // rt,),
        in_specs=[
            pl.BlockSpec((rt, d), lambda t: (t, 0)),
            pl.BlockSpec((n_experts, d), lambda t: (0, 0)),
            pl.BlockSpec((1, n_experts), lambda t: (0, 0)),
        ],
        out_specs=[
            pl.BlockSpec((rt, 2), lambda t: (t, 0)),
            pl.BlockSpec((rt, 2), lambda t: (t, 0)),
            pl.BlockSpec((1, n_experts), lambda t: (0, 0)),
            pl.BlockSpec((1, 1), lambda t: (0, 0)),
        ],
        out_shape=[
            jax.ShapeDtypeStruct((tokens, 2), jnp.int32),
            jax.ShapeDtypeStruct((tokens, 2), jnp.float32),
            jax.ShapeDtypeStruct((1, n_experts), jnp.float32),
            jax.ShapeDtypeStruct((1, 1), jnp.float32),
        ],
    )(x_flat, gate_W, gate_b.reshape(1, n_experts))

    eflat = ew.T.reshape(n_assign // 128, 128)

    pos, te = pl.pallas_call(
        functools.partial(_route_pos_body, n_assign=n_assign,
                          n_experts=n_experts, n_tiles=n_tiles),
        out_shape=[
            jax.ShapeDtypeStruct((n_assign // 128, 128), jnp.int32),
            jax.ShapeDtypeStruct((1, 128), jnp.int32),
        ],
    )(eflat)

    pos2d = pos.reshape(128, 64)

    mesh = plsc.VectorSubcoreMesh(core_axis_name="c", subcore_axis_name="s")
    xs_sorted = pl.kernel(
        _sc_scatter_body,
        out_type=jax.ShapeDtypeStruct((cap, d), jnp.float32),
        mesh=mesh,
        scratch_types=[
            pltpu.VMEM((4, 64), jnp.int32),
            pltpu.VMEM((64, d), jnp.float32),
            pltpu.SemaphoreType.DMA,
        ],
    )(x_flat, pos2d)

    dffh = dff // 2
    b1r = b1.reshape(n_experts, 1, dff)
    b2r = b2.reshape(n_experts, 1, d)

    def _half_spec(f):
        return pltpu.PrefetchScalarGridSpec(
            num_scalar_prefetch=1,
            grid=(n_tiles,),
            in_specs=[
                pl.BlockSpec((_TILE, d), lambda t, te_r: (t, 0)),
                pl.BlockSpec((1, dffh, d), lambda t, te_r: (te_r[0, t], f, 0)),
                pl.BlockSpec((1, 1, dffh), lambda t, te_r: (te_r[0, t], 0, f)),
                pl.BlockSpec((1, d, dffh), lambda t, te_r: (te_r[0, t], 0, f)),
                pl.BlockSpec((1, 1, d), lambda t, te_r: (te_r[0, t], 0, 0)),
            ],
            out_specs=pl.BlockSpec((_TILE, d), lambda t, te_r: (t, 0)),
        )

    ys_a = pl.pallas_call(
        functools.partial(_ffn_half_body, use_b2=True),
        grid_spec=_half_spec(0),
        out_shape=jax.ShapeDtypeStruct((cap, d), jnp.float32),
    )(te, xs_sorted, W1, b1r, W2, b2r)
    ys_b = pl.pallas_call(
        functools.partial(_ffn_half_body, use_b2=False),
        grid_spec=_half_spec(1),
        out_shape=jax.ShapeDtypeStruct((cap, d), jnp.float32),
    )(te, xs_sorted, W1, b1r, W2, b2r)

    rows_a, rows_b = pl.kernel(
        _sc_gather_body,
        out_type=(jax.ShapeDtypeStruct((n_assign, d), jnp.float32),
                  jax.ShapeDtypeStruct((n_assign, d), jnp.float32)),
        mesh=mesh,
        scratch_types=[
            pltpu.VMEM((4, 64), jnp.int32),
            pltpu.VMEM((64, d), jnp.float32),
            pltpu.SemaphoreType.DMA,
        ],
    )(ys_a, ys_b, pos2d)

    ct = 512
    out_flat = pl.pallas_call(
        _combine_body,
        grid=(tokens // ct,),
        in_specs=[
            pl.BlockSpec((ct, d), lambda t: (t, 0)),
            pl.BlockSpec((ct, d), lambda t: (t + tokens // ct, 0)),
            pl.BlockSpec((ct, d), lambda t: (t, 0)),
            pl.BlockSpec((ct, d), lambda t: (t + tokens // ct, 0)),
            pl.BlockSpec((ct, 1), lambda t: (t, 0)),
            pl.BlockSpec((ct, 1), lambda t: (t, 0)),
        ],
        out_specs=pl.BlockSpec((ct, d), lambda t: (t, 0)),
        out_shape=jax.ShapeDtypeStruct((tokens, d), jnp.float32),
    )(rows_a, rows_a, rows_b, rows_b, wv[:, 0:1], wv[:, 1:2])

    return (out_flat.reshape(seq_len, batch, d), loss.reshape(()))


# speedup vs baseline: 5.4750x; 1.0325x over previous
import functools

import jax
import jax.numpy as jnp
from jax import lax
from jax.experimental import pallas as pl
from jax.experimental.pallas import tpu as pltpu
from jax.experimental.pallas import tpu_sc as plsc

_NC = 2
_NS = 16
_TILE = 256


def _router_body(x_ref, gw_ref, gb_ref, e_ref, w_ref, psum_ref, loss_ref, *,
                 n_tokens, n_experts):
    t = pl.program_id(0)
    xs = x_ref[...]
    gw = gw_ref[...]
    logits = lax.dot_general(xs, gw, (((1,), (1,)), ((), ())),
                             preferred_element_type=jnp.float32) + gb_ref[...]
    m = jnp.max(logits, axis=1, keepdims=True)
    p = jnp.exp(logits - m)
    probs = p / jnp.sum(p, axis=1, keepdims=True)
    iota = lax.broadcasted_iota(jnp.int32, probs.shape, 1)
    m1 = jnp.max(probs, axis=1, keepdims=True)
    i0 = jnp.min(jnp.where(probs == m1, iota, n_experts), axis=1, keepdims=True)
    probs2 = jnp.where(iota == i0, -1.0, probs)
    m2 = jnp.max(probs2, axis=1, keepdims=True)
    i1 = jnp.min(jnp.where(probs2 == m2, iota, n_experts), axis=1, keepdims=True)
    e_ref[...] = jnp.concatenate([i0, i1], axis=1)
    w_ref[...] = jnp.concatenate([m1, m2], axis=1)

    @pl.when(t == 0)
    def _():
        psum_ref[...] = jnp.zeros_like(psum_ref)

    psum_ref[...] += jnp.sum(probs, axis=0, keepdims=True)

    @pl.when(t == pl.num_programs(0) - 1)
    def _():
        mean = psum_ref[...] * (1.0 / n_tokens)
        loss_ref[...] = jnp.sum(mean * mean, axis=1, keepdims=True) * n_experts


def _route_pos_body(e_ref, pos_ref, te_ref, used_ref, *, n_assign, n_experts,
                    n_tiles):
    blk = 128
    n_rows = n_assign // blk
    ev = e_ref[...]
    rb = lax.broadcasted_iota(jnp.int32, (blk, blk), 0)
    cb = lax.broadcasted_iota(jnp.int32, (blk, blk), 1)
    ustrict = (rb < cb).astype(jnp.float32)
    rr = lax.broadcasted_iota(jnp.int32, (n_rows, n_rows), 0)
    cr = lax.broadcasted_iota(jnp.int32, (n_rows, n_rows), 1)
    lstrict = (cr < rr).astype(jnp.float32)
    iota_e = lax.broadcasted_iota(jnp.int32, (1, n_experts), 1)

    rank = jnp.zeros((n_rows, blk), jnp.float32)
    totals = jnp.zeros((1, n_experts), jnp.float32)
    ms = []
    for e in range(n_experts):
        m = (ev == e).astype(jnp.float32)
        ms.append(m)
        lane_pre = lax.dot_general(m, ustrict, (((1,), (0,)), ((), ())),
                                   preferred_element_type=jnp.float32)
        s = jnp.sum(m, axis=1, keepdims=True)
        row_pre = lax.dot_general(lstrict, s, (((1,), (0,)), ((), ())),
                                  preferred_element_type=jnp.float32)
        rank = rank + m * (lane_pre + row_pre)
        totals = totals + jnp.sum(s, axis=0, keepdims=True) * (
            iota_e == e).astype(jnp.float32)

    padded = jnp.ceil(totals * (1.0 / _TILE)) * _TILE
    re = lax.broadcasted_iota(jnp.int32, (n_experts, n_experts), 0)
    ce = lax.broadcasted_iota(jnp.int32, (n_experts, n_experts), 1)
    lower_inc = (re <= ce).astype(jnp.float32)
    incl = lax.dot_general(padded, lower_inc, (((1,), (0,)), ((), ())),
                           preferred_element_type=jnp.float32)
    base = incl - padded

    basesel = jnp.zeros((n_rows, blk), jnp.float32)
    for e in range(n_experts):
        basesel = basesel + ms[e] * base[0:1, e:e + 1]
    pos_ref[...] = (rank + basesel).astype(jnp.int32)

    t_ids = lax.broadcasted_iota(jnp.int32, (1, 128), 1)
    end_tiles = (incl * (1.0 / _TILE)).astype(jnp.int32)
    te = jnp.zeros((1, 128), jnp.int32)
    for e in range(n_experts):
        end_e = end_tiles[0:1, e:e + 1]
        te = te + (t_ids >= end_e).astype(jnp.int32)
    te_ref[...] = jnp.minimum(te, n_experts - 1)
    used_ref[...] = end_tiles[0:1, n_experts - 1:n_experts]


def _sc_scatter_body(x_hbm, pos_hbm, xs_hbm, posv, xbuf, sem):
    wid = lax.axis_index("s") * _NC + lax.axis_index("c")
    tb = lax.rem(wid * 256, 4096)
    pltpu.sync_copy(pos_hbm.at[pl.ds(wid * 4, 4)], posv)
    for c in range(4):
        pltpu.sync_copy(x_hbm.at[pl.ds(tb + c * 64, 64)], xbuf)
        pltpu.async_copy(xbuf, xs_hbm.at[posv.at[c]], sem).wait()


def _sc_gather_body(ya_hbm, yb_hbm, pos_hbm, ra_hbm, rb_hbm, posv, xbuf, sem):
    wid = lax.axis_index("s") * _NC + lax.axis_index("c")
    pltpu.sync_copy(pos_hbm.at[pl.ds(wid * 4, 4)], posv)
    for c in range(4):
        pltpu.async_copy(ya_hbm.at[posv.at[c]], xbuf, sem).wait()
        pltpu.sync_copy(xbuf, ra_hbm.at[pl.ds(wid * 256 + c * 64, 64)])
        pltpu.async_copy(yb_hbm.at[posv.at[c]], xbuf, sem).wait()
        pltpu.sync_copy(xbuf, rb_hbm.at[pl.ds(wid * 256 + c * 64, 64)])


def _ffn_half_body(te_ref, used_ref, xs_ref, w1_ref, b1_ref, w2_ref, b2_ref,
                   out_ref, *, use_b2):
    del te_ref

    @pl.when(pl.program_id(0) < used_ref[0, 0])
    def _():
        h = lax.dot_general(xs_ref[...], w1_ref[0], (((1,), (1,)), ((), ())),
                            preferred_element_type=jnp.float32)
        h = jnp.maximum(h + b1_ref[0], 0.0)
        proc = lax.dot_general(h, w2_ref[0], (((1,), (1,)), ((), ())),
                               preferred_element_type=jnp.float32)
        out_ref[...] = proc + b2_ref[0] if use_b2 else proc


def _combine_body(a0_ref, a1_ref, b0_ref, b1_ref, w0_ref, w1_ref, out_ref):
    out_ref[...] = ((a0_ref[...] + b0_ref[...]) * w0_ref[...] +
                    (a1_ref[...] + b1_ref[...]) * w1_ref[...])


def kernel(x, gate_W, gate_b, W1, b1, W2, b2):
    seq_len, batch, d = x.shape
    n_experts, dff, _ = W1.shape
    tokens = seq_len * batch
    n_assign = 2 * tokens
    cap = n_assign + n_experts * _TILE
    n_tiles = cap // _TILE
    x_flat = x.reshape(tokens, d)

    rt = 512
    ew, wv, _, loss = pl.pallas_call(
        functools.partial(_router_body, n_tokens=tokens, n_experts=n_experts),
        grid=(tokens // rt,),
        in_specs=[
            pl.BlockSpec((rt, d), lambda t: (t, 0)),
            pl.BlockSpec((n_experts, d), lambda t: (0, 0)),
            pl.BlockSpec((1, n_experts), lambda t: (0, 0)),
        ],
        out_specs=[
            pl.BlockSpec((rt, 2), lambda t: (t, 0)),
            pl.BlockSpec((rt, 2), lambda t: (t, 0)),
            pl.BlockSpec((1, n_experts), lambda t: (0, 0)),
            pl.BlockSpec((1, 1), lambda t: (0, 0)),
        ],
        out_shape=[
            jax.ShapeDtypeStruct((tokens, 2), jnp.int32),
            jax.ShapeDtypeStruct((tokens, 2), jnp.float32),
            jax.ShapeDtypeStruct((1, n_experts), jnp.float32),
            jax.ShapeDtypeStruct((1, 1), jnp.float32),
        ],
    )(x_flat, gate_W, gate_b.reshape(1, n_experts))

    eflat = ew.T.reshape(n_assign // 128, 128)

    pos, te, used = pl.pallas_call(
        functools.partial(_route_pos_body, n_assign=n_assign,
                          n_experts=n_experts, n_tiles=n_tiles),
        out_shape=[
            jax.ShapeDtypeStruct((n_assign // 128, 128), jnp.int32),
            jax.ShapeDtypeStruct((1, 128), jnp.int32),
            jax.ShapeDtypeStruct((1, 1), jnp.int32),
        ],
    )(eflat)

    pos2d = pos.reshape(128, 64)

    mesh = plsc.VectorSubcoreMesh(core_axis_name="c", subcore_axis_name="s")
    xs_sorted = pl.kernel(
        _sc_scatter_body,
        out_type=jax.ShapeDtypeStruct((cap, d), jnp.float32),
        mesh=mesh,
        scratch_types=[
            pltpu.VMEM((4, 64), jnp.int32),
            pltpu.VMEM((64, d), jnp.float32),
            pltpu.SemaphoreType.DMA,
        ],
    )(x_flat, pos2d)

    dffh = dff // 2
    b1r = b1.reshape(n_experts, 1, dff)
    b2r = b2.reshape(n_experts, 1, d)

    def _half_spec(f):
        return pltpu.PrefetchScalarGridSpec(
            num_scalar_prefetch=2,
            grid=(n_tiles,),
            in_specs=[
                pl.BlockSpec((_TILE, d), lambda t, te_r, u_r: (t, 0)),
                pl.BlockSpec((1, dffh, d),
                             lambda t, te_r, u_r: (te_r[0, t], f, 0)),
                pl.BlockSpec((1, 1, dffh),
                             lambda t, te_r, u_r: (te_r[0, t], 0, f)),
                pl.BlockSpec((1, d, dffh),
                             lambda t, te_r, u_r: (te_r[0, t], 0, f)),
                pl.BlockSpec((1, 1, d),
                             lambda t, te_r, u_r: (te_r[0, t], 0, 0)),
            ],
            out_specs=pl.BlockSpec((_TILE, d), lambda t, te_r, u_r: (t, 0)),
        )

    ys_a = pl.pallas_call(
        functools.partial(_ffn_half_body, use_b2=True),
        grid_spec=_half_spec(0),
        out_shape=jax.ShapeDtypeStruct((cap, d), jnp.float32),
    )(te, used, xs_sorted, W1, b1r, W2, b2r)
    ys_b = pl.pallas_call(
        functools.partial(_ffn_half_body, use_b2=False),
        grid_spec=_half_spec(1),
        out_shape=jax.ShapeDtypeStruct((cap, d), jnp.float32),
    )(te, used, xs_sorted, W1, b1r, W2, b2r)

    rows_a, rows_b = pl.kernel(
        _sc_gather_body,
        out_type=(jax.ShapeDtypeStruct((n_assign, d), jnp.float32),
                  jax.ShapeDtypeStruct((n_assign, d), jnp.float32)),
        mesh=mesh,
        scratch_types=[
            pltpu.VMEM((4, 64), jnp.int32),
            pltpu.VMEM((64, d), jnp.float32),
            pltpu.SemaphoreType.DMA,
        ],
    )(ys_a, ys_b, pos2d)

    ct = 512
    out_flat = pl.pallas_call(
        _combine_body,
        grid=(tokens // ct,),
        in_specs=[
            pl.BlockSpec((ct, d), lambda t: (t, 0)),
            pl.BlockSpec((ct, d), lambda t: (t + tokens // ct, 0)),
            pl.BlockSpec((ct, d), lambda t: (t, 0)),
            pl.BlockSpec((ct, d), lambda t: (t + tokens // ct, 0)),
            pl.BlockSpec((ct, 1), lambda t: (t, 0)),
            pl.BlockSpec((ct, 1), lambda t: (t, 0)),
        ],
        out_specs=pl.BlockSpec((ct, d), lambda t: (t, 0)),
        out_shape=jax.ShapeDtypeStruct((tokens, d), jnp.float32),
    )(rows_a, rows_a, rows_b, rows_b, wv[:, 0:1], wv[:, 1:2])

    return (out_flat.reshape(seq_len, batch, d), loss.reshape(()))
